# Initial kernel scaffold; baseline (speedup 1.0000x reference)
import jax
import jax.numpy as jnp
from jax.experimental import pallas as pl


def kernel(input_ids, embed_weight, proj_weight, proj_bias):
    raise NotImplementedError("write your pallas kernel here")


# speedup vs baseline: 1.9627x; 1.9627x over previous
import functools

import jax
import jax.numpy as jnp
from jax import lax
from jax.experimental import pallas as pl
from jax.experimental.pallas import tpu as pltpu
from jax.experimental.pallas import tpu_sc as plsc

VOCAB_SIZE = 100
EMB_DIM = 64
CHUNK = 128


def _table_body(emb_ref, projt_ref, bias_ref, out_ref):
    out_ref[...] = (
        jnp.dot(emb_ref[...], projt_ref[...], preferred_element_type=jnp.float32)
        + bias_ref[...]
    )


def _make_table(embed_weight, proj_weight, proj_bias):
    return pl.pallas_call(
        _table_body,
        out_shape=jax.ShapeDtypeStruct((VOCAB_SIZE, VOCAB_SIZE), jnp.float32),
    )(embed_weight, proj_weight.T, proj_bias.reshape(1, VOCAB_SIZE))


def _sc_gather(table, ids3, width):
    nw, nchunk, cw = ids3.shape
    info = plsc.get_sparse_core_info()
    nc = info.num_cores

    mesh = plsc.VectorSubcoreMesh(core_axis_name="c", subcore_axis_name="s")

    @functools.partial(
        pl.kernel,
        mesh=mesh,
        out_type=jax.ShapeDtypeStruct((nw * nchunk, cw, width), jnp.float32),
        scratch_types=[
            pltpu.VMEM((nchunk, cw), jnp.int32),
            pltpu.VMEM((cw, width), jnp.float32),
            pltpu.SemaphoreType.DMA,
        ],
        compiler_params=pltpu.CompilerParams(use_tc_tiling_on_sc=False),
    )
    def gather_kernel(table_hbm, idx_hbm, out_hbm, idx_v, rows_v, sem):
        wid = lax.axis_index("s") * nc + lax.axis_index("c")
        pltpu.sync_copy(idx_hbm.at[wid], idx_v)

        def body(j, carry):
            pltpu.async_copy(table_hbm.at[idx_v.at[j]], rows_v, sem).wait()
            pltpu.sync_copy(rows_v, out_hbm.at[wid * nchunk + j])
            return carry

        lax.fori_loop(0, nchunk, body, 0)

    return gather_kernel(table, ids3)


def kernel(input_ids, embed_weight, proj_weight, proj_bias):
    b, l = input_ids.shape
    n_tok = b * l
    nw = 32
    table = _make_table(embed_weight, proj_weight, proj_bias)
    ids3 = input_ids.reshape(nw, n_tok // (nw * CHUNK), CHUNK).astype(jnp.int32)
    table128 = jnp.pad(table, ((0, 0), (0, 104 - VOCAB_SIZE)))
    out = _sc_gather(table128, ids3, 104)
    return out[:, :, :VOCAB_SIZE].reshape(b, l, VOCAB_SIZE)


# speedup vs baseline: 2.2380x; 1.1403x over previous
import functools

import jax
import jax.numpy as jnp
from jax import lax
from jax.experimental import pallas as pl
from jax.experimental.pallas import tpu as pltpu
from jax.experimental.pallas import tpu_sc as plsc

VOCAB_SIZE = 100
EMB_DIM = 64
PAIR_W = 2 * VOCAB_SIZE
CHUNK = 256


def _table_body(emb_ref, projt_ref, bias_ref, out_ref):
    out_ref[...] = (
        jnp.dot(emb_ref[...], projt_ref[...], preferred_element_type=jnp.float32)
        + bias_ref[...]
    )


def _make_table(embed_weight, proj_weight, proj_bias):
    return pl.pallas_call(
        _table_body,
        out_shape=jax.ShapeDtypeStruct((VOCAB_SIZE, VOCAB_SIZE), jnp.float32),
    )(embed_weight, proj_weight.T, proj_bias.reshape(1, VOCAB_SIZE))


def _pair_body(table_ref, out_ref):
    t = table_ref[...]
    v = VOCAB_SIZE
    out_ref[:, :, 0, :] = jnp.broadcast_to(t[:, None, :], (v, v, v))
    out_ref[:, :, 1, :] = jnp.broadcast_to(t[None, :, :], (v, v, v))


def _make_pair_table(table):
    v = VOCAB_SIZE
    out = pl.pallas_call(
        _pair_body,
        out_shape=jax.ShapeDtypeStruct((v, v, 2, v), jnp.float32),
    )(table)
    return out.reshape(v * v, PAIR_W)


def _sc_gather(pair_table, ids3):
    nw, nchunk, cw = ids3.shape
    npair = cw // 2
    info = plsc.get_sparse_core_info()
    nc = info.num_cores

    mesh = plsc.VectorSubcoreMesh(core_axis_name="c", subcore_axis_name="s")

    @functools.partial(
        pl.kernel,
        mesh=mesh,
        out_type=jax.ShapeDtypeStruct((nw * nchunk, npair, PAIR_W), jnp.float32),
        scratch_types=[
            pltpu.VMEM((nchunk, cw), jnp.int32),
            pltpu.VMEM((nchunk, npair), jnp.int32),
            pltpu.VMEM((npair, PAIR_W), jnp.float32),
            pltpu.SemaphoreType.DMA,
        ],
        compiler_params=pltpu.CompilerParams(
            use_tc_tiling_on_sc=False, needs_layout_passes=False
        ),
    )
    def gather_kernel(ptab_hbm, idx_hbm, out_hbm, idx_v, pidx_v, rows_v, sem):
        wid = lax.axis_index("s") * nc + lax.axis_index("c")
        pltpu.sync_copy(idx_hbm.at[wid], idx_v)

        lane = lax.iota(jnp.int32, 16)

        def make_pairs(j, carry):
            jv = jnp.full((16,), j, dtype=jnp.int32)
            for s in range(npair // 16):
                pos = lane * 2 + (32 * s)
                a = plsc.load_gather(idx_v, [jv, pos])
                b = plsc.load_gather(idx_v, [jv, pos + 1])
                pidx_v[j, pl.ds(16 * s, 16)] = a * VOCAB_SIZE + b
            return carry

        lax.fori_loop(0, nchunk, make_pairs, 0)

        def body(j, carry):
            pltpu.async_copy(ptab_hbm.at[pidx_v.at[j]], rows_v, sem).wait()
            pltpu.sync_copy(rows_v, out_hbm.at[wid * nchunk + j])
            return carry

        lax.fori_loop(0, nchunk, body, 0)

    return gather_kernel(pair_table, ids3)


def kernel(input_ids, embed_weight, proj_weight, proj_bias):
    b, l = input_ids.shape
    n_tok = b * l
    nw = 32
    table = _make_table(embed_weight, proj_weight, proj_bias)
    pair_table = _make_pair_table(table)
    ids3 = input_ids.reshape(nw, n_tok // (nw * CHUNK), CHUNK).astype(jnp.int32)
    out = _sc_gather(pair_table, ids3)
    return out.reshape(b, l, VOCAB_SIZE)


# speedup vs baseline: 20.2454x; 9.0460x over previous
import jax
import jax.numpy as jnp
from jax import lax
from jax.experimental import pallas as pl

VOCAB_SIZE = 100
EMB_DIM = 64
BL = 8
BB = 2048


def _tablet_body(proj_ref, embt_ref, bias_ref, out_ref):
    out_ref[...] = (
        jnp.dot(proj_ref[...], embt_ref[...], preferred_element_type=jnp.float32)
        + bias_ref[...]
    )


def _make_tablet(embed_weight, proj_weight, proj_bias):
    return pl.pallas_call(
        _tablet_body,
        out_shape=jax.ShapeDtypeStruct((VOCAB_SIZE, VOCAB_SIZE), jnp.float32),
    )(proj_weight, embed_weight.T, proj_bias.reshape(VOCAB_SIZE, 1))


def _logits_body(tabt_ref, ids_ref, out_ref):
    tabt = tabt_ref[...]
    for li in range(BL):
        ids_row = ids_ref[li, :]
        u = lax.broadcasted_iota(jnp.int32, (VOCAB_SIZE, BB), 0)
        onehot = (u == ids_row[None, :]).astype(jnp.float32)
        out_ref[:, li, :] = jnp.dot(
            tabt, onehot, preferred_element_type=jnp.float32
        )


def _logits_t(tablet, ids_lb):
    l, b = ids_lb.shape
    grid = (l // BL, b // BB)
    return pl.pallas_call(
        _logits_body,
        grid=grid,
        in_specs=[
            pl.BlockSpec((VOCAB_SIZE, VOCAB_SIZE), lambda i, j: (0, 0)),
            pl.BlockSpec((BL, BB), lambda i, j: (i, j)),
        ],
        out_specs=pl.BlockSpec((VOCAB_SIZE, BL, BB), lambda i, j: (0, i, j)),
        out_shape=jax.ShapeDtypeStruct((VOCAB_SIZE, l, b), jnp.float32),
    )(tablet, ids_lb)


def kernel(input_ids, embed_weight, proj_weight, proj_bias):
    tablet = _make_tablet(embed_weight, proj_weight, proj_bias)
    ids_lb = input_ids.T.astype(jnp.int32)
    out_t = _logits_t(tablet, ids_lb)
    return out_t.transpose(2, 1, 0)
